# baseline (speedup 1.0000x reference)
import functools

import jax
import jax.numpy as jnp
from jax import lax
from jax.experimental import pallas as pl
from jax.experimental.pallas import tpu as pltpu
from jax.experimental.pallas import tpu_sc as plsc

N = 50000
E = 800000
EMB = 96
H = EMB // 3

NC = 2
NS = 16
G = 128

SLC = 3
SW = EMB // SLC
SPT = N // NS
DST_PAD = N
NPo = 50176

E_PAD = 802816
EROWS = E_PAD // G
RPT_D = EROWS // (NC * NS)
SCH_R = 28
NSCH = RPT_D // SCH_R
GR = 2
NG2 = SCH_R // GR
GE = GR * G

NUP = 50176
ZSH = 51712
ZPT = ZSH // NS

_MESH = plsc.VectorSubcoreMesh(core_axis_name="c", subcore_axis_name="s")


def _leaky(x):
    return jnp.where(x > 0, x, 0.01 * x)


@functools.partial(
    pl.kernel,
    out_type=jax.ShapeDtypeStruct((NC * NUP,), jnp.float32),
    mesh=_MESH,
    scratch_types=[
        pltpu.VMEM_SHARED((ZSH,), jnp.float32),
        pltpu.VMEM((ZPT,), jnp.float32),
        pltpu.VMEM((G,), jnp.float32),
        pltpu.VMEM((4, G), jnp.int32),
    ],
)
def _deg_sc(dst_hbm, out_hbm, deg_sh, zbuf, oneb, idxb):
    c = lax.axis_index("c")
    s = lax.axis_index("s")
    for i in range(ZPT // 16):
        zbuf[pl.ds(i * 16, 16)] = jnp.zeros((16,), jnp.float32)
    pltpu.sync_copy(zbuf, deg_sh.at[pl.ds(s * ZPT, ZPT)])
    for i in range(G // 16):
        oneb[pl.ds(i * 16, 16)] = jnp.ones((16,), jnp.float32)
    plsc.subcore_barrier()

    base = (c * NS + s) * RPT_D
    def body(i, carry):
        pltpu.sync_copy(dst_hbm.at[pl.ds(base + i * 4, 4)], idxb)
        for j in range(4):
            pltpu.sync_copy(oneb, deg_sh.at[idxb.at[j]], add=True)
        return carry
    lax.fori_loop(0, RPT_D // 4, body, 0)
    plsc.subcore_barrier()

    span = NUP // NS
    pltpu.sync_copy(deg_sh.at[pl.ds(s * span, span)], zbuf.at[pl.ds(0, span)])
    pltpu.sync_copy(zbuf.at[pl.ds(0, span)],
                    out_hbm.at[pl.ds(c * NUP + s * span, span)])


@functools.partial(
    pl.kernel,
    out_type=jax.ShapeDtypeStruct((NC * SLC * NPo, SW), jnp.float32),
    mesh=_MESH,
    scratch_types=[
        pltpu.VMEM_SHARED((N + 8, SW), jnp.float32),
        pltpu.VMEM((SCH_R * G,), jnp.int32),
        pltpu.VMEM((SCH_R * G,), jnp.int32),
        pltpu.VMEM((GE, SW), jnp.float32),
        pltpu.VMEM((GE, SW), jnp.float32),
        pltpu.SemaphoreType.DMA,
        pltpu.SemaphoreType.DMA,
        pltpu.SemaphoreType.DMA,
    ],
    compiler_params=pltpu.CompilerParams(use_tc_tiling_on_sc=False),
)
def _conv_sc(hp0, hp1, hp2, src_hbm, dst_hbm, out_hbm,
             acc_sh, srcb, dstb, gb0, gb1, sem0, sem1, ssem):
    c = lax.axis_index("c")
    s = lax.axis_index("s")
    gbufs = (gb0, gb1)
    sems = (sem0, sem1)
    start_g = s * SPT
    rowbase = c * (EROWS // NC) + s * RPT_D

    for p, hp in enumerate((hp0, hp1, hp2)):
        pltpu.sync_copy(hp.at[pl.ds(start_g, SPT)],
                        acc_sh.at[pl.ds(start_g, SPT)])
        plsc.subcore_barrier()

        def schunk(si, carry):
            base = (rowbase + si * SCH_R) * G
            pltpu.sync_copy(src_hbm.at[pl.ds(base, SCH_R * G)], srcb)
            pltpu.sync_copy(dst_hbm.at[pl.ds(base, SCH_R * G)], dstb)
            gds = [None, None]
            gds[0] = pltpu.async_copy(
                hp.at[srcb.at[pl.ds(0, GE)]], gb0, sem0)
            for k in range(NG2):
                b = k % 2
                gds[b].wait()
                if k + 1 < NG2:
                    gds[1 - b] = pltpu.async_copy(
                        hp.at[srcb.at[pl.ds((k + 1) * GE, GE)]],
                        gbufs[1 - b], sems[1 - b])
                pltpu.async_copy(
                    gbufs[b], acc_sh.at[dstb.at[pl.ds(k * GE, GE)]],
                    ssem, add=True).wait()
            return carry
        lax.fori_loop(0, NSCH, schunk, 0)
        plsc.subcore_barrier()
        q = c * SLC + p
        pltpu.sync_copy(acc_sh.at[pl.ds(start_g, SPT)],
                        out_hbm.at[pl.ds(q * NPo + start_g, SPT)])
        plsc.subcore_barrier()


RB = 1000
GRID = N // RB


def _mm(a, b):
    return jnp.dot(a, b, preferred_element_type=jnp.float32)


def _enc_body(des_b, num_b, cat_b, Wd_b, bd_b, Wn_b, bn_b, Wc_b, bc_b,
              Wi_b, bi_b, out_b):
    d = _leaky(_mm(des_b[...], Wd_b[...]) + bd_b[...])
    n = _leaky(_mm(num_b[...], Wn_b[...]) + bn_b[...])
    cc = _leaky(_mm(cat_b[...], Wc_b[...]) + bc_b[...])
    x = jnp.concatenate([d, n, cc], axis=1)
    out_b[...] = _leaky(_mm(x, Wi_b[...]) + bi_b[...])


def _dis_of(degs_b):
    dd = degs_b[...]
    return lax.rsqrt(1.0 + dd[:, 0] + dd[:, 1])


def _split3(h, o0, o1, o2):
    o0[...] = h[:, 0:SW]
    o1[...] = h[:, SW:2 * SW]
    o2[...] = h[:, 2 * SW:EMB]


def _h1_body(x1_b, degs_b, Wg1_b, o0, o1, o2, od):
    dis = _dis_of(degs_b)
    od[...] = dis[:, None]
    _split3(_mm(x1_b[...], Wg1_b[...]) * dis[:, None], o0, o1, o2)


def _pack_x(parts, hps, df, bf):
    return [
        (parts[p][...] + parts[SLC + p][...] - hps[p][...]) * df + bf[p][...]
        for p in range(SLC)]


def _yf_slice(yf, q, df):
    return jnp.concatenate(
        [yf[:, 96 * a + SW * q:96 * a + SW * q + SW] for a in range(4)],
        axis=1) * df


def _h2_body(a0, a1, a2, a3, a4, a5, h0, h1, h2, df_b, bf0, bf1, bf2,
             W0, W1, W2, o0, o1, o2):
    df = df_b[...]
    xfs = _pack_x((a0, a1, a2, a3, a4, a5), (h0, h1, h2), df,
                  (bf0, bf1, bf2))
    yf = _mm(xfs[0], W0[...]) + _mm(xfs[1], W1[...]) + _mm(xfs[2], W2[...])
    o0[...] = _yf_slice(yf, 0, df)
    o1[...] = _yf_slice(yf, 1, df)
    o2[...] = _yf_slice(yf, 2, df)


def _head_body(a0, a1, a2, a3, a4, a5, h0, h1, h2, df_b, bf0, bf1, bf2,
               W0, W1, W2, bo1f_b, Wo2f_b, bo2f_b, out_b):
    df = df_b[...]
    xfs = _pack_x((a0, a1, a2, a3, a4, a5), (h0, h1, h2), df,
                  (bf0, bf1, bf2))
    yf = _leaky(_mm(xfs[0], W0[...]) + _mm(xfs[1], W1[...])
                + _mm(xfs[2], W2[...]) + bo1f_b[...])
    out_b[...] = _mm(yf, Wo2f_b[...]) + bo2f_b[...]


def _row_spec(cols):
    return pl.BlockSpec((RB, cols), lambda i: (i, 0))


def _full_spec(shape):
    return pl.BlockSpec(shape, lambda i: tuple(0 for _ in shape))


def _deg_spec():
    return pl.BlockSpec((RB, 2), lambda i: (i, 0))


FRB = 256
FGRID = NPo // (4 * FRB)
FQ = NPo // (4 * FRB)


def _flat_spec():
    return pl.BlockSpec((FRB, 4 * SW), lambda i: (i, 0))


def _part_spec(q):
    return pl.BlockSpec((FRB, 4 * SW), lambda i, q=q: (q * FQ + i, 0))


def _part_specs():
    return [_part_spec(q) for q in range(NC * SLC)]


def _kron4(w):
    return jnp.kron(jnp.eye(4, dtype=w.dtype), w)


def kernel(des, tweet, num_prop, cat_prop, edge_index,
           Wd, bd, Wn, bn, Wc, bc, Wi, bi, Wg1, bg1, Wg2, bg2,
           Wo1, bo1, Wo2, bo2):
    del tweet
    f32 = jnp.float32
    src = edge_index[0]
    dst = edge_index[1]
    pad = E_PAD - E
    src_p = jnp.concatenate([src, jnp.zeros((pad,), jnp.int32)])
    dst_p = jnp.concatenate(
        [dst, jnp.full((pad,), DST_PAD, jnp.int32)])

    degs = _deg_sc(dst_p.reshape(EROWS, G)).reshape(NC, NUP)[:, :N].T

    x1 = pl.pallas_call(
        _enc_body,
        grid=(GRID,),
        in_specs=[
            _row_spec(768), _row_spec(4), _row_spec(3),
            _full_spec((768, H)), _full_spec((1, H)),
            _full_spec((4, H)), _full_spec((1, H)),
            _full_spec((3, H)), _full_spec((1, H)),
            _full_spec((EMB, EMB)), _full_spec((1, EMB)),
        ],
        out_specs=_row_spec(EMB),
        out_shape=jax.ShapeDtypeStruct((N, EMB), f32),
    )(des, num_prop, cat_prop, Wd, bd.reshape(1, H), Wn, bn.reshape(1, H),
      Wc, bc.reshape(1, H), Wi, bi.reshape(1, EMB))

    h1out = pl.pallas_call(
        _h1_body,
        grid=(GRID,),
        in_specs=[_row_spec(EMB), _deg_spec(), _full_spec((EMB, EMB))],
        out_specs=[_row_spec(SW)] * SLC + [_row_spec(1)],
        out_shape=[jax.ShapeDtypeStruct((N, SW), f32)] * SLC
        + [jax.ShapeDtypeStruct((N, 1), f32)],
    )(x1, degs, Wg1)
    h1, dis_nd = h1out[:SLC], h1out[SLC]

    disq = jnp.concatenate([dis_nd, jnp.ones((NPo - N, 1), f32)])
    dis_f = jnp.broadcast_to(
        disq.reshape(NPo // 4, 4, 1), (NPo // 4, 4, SW)).reshape(NPo // 4,
                                                                 4 * SW)
    w2f = [_kron4(Wg2[SW * p:SW * (p + 1), :]) for p in range(SLC)]
    wo1f = [_kron4(Wo1[SW * p:SW * (p + 1), :]) for p in range(SLC)]
    wo2f = _kron4(Wo2)
    bg1f = [jnp.tile(bg1[SW * p:SW * (p + 1)], 4).reshape(1, 4 * SW)
            for p in range(SLC)]
    bg2f = [jnp.tile(bg2[SW * p:SW * (p + 1)], 4).reshape(1, 4 * SW)
            for p in range(SLC)]
    bo1f = jnp.tile(bo1, 4).reshape(1, 4 * EMB)
    bo2f = jnp.tile(bo2, 4).reshape(1, 8)

    flat_out = [jax.ShapeDtypeStruct((NPo // 4, 4 * SW), f32)] * SLC
    fspecs = ([_flat_spec()] * SLC
              + [_flat_spec(), _full_spec((1, 4 * SW)),
                 _full_spec((1, 4 * SW)), _full_spec((1, 4 * SW)),
                 _full_spec((4 * SW, 4 * EMB)), _full_spec((4 * SW, 4 * EMB)),
                 _full_spec((4 * SW, 4 * EMB))])

    h1f = [h.reshape(N // 4, 4 * SW) for h in h1]
    t1 = _conv_sc(*(h.reshape(N, SW) for h in h1f), src_p, dst_p)
    t1f = t1.reshape(NC * SLC * NPo // 4, 4 * SW)

    h2 = pl.pallas_call(
        _h2_body,
        grid=(FGRID,),
        in_specs=_part_specs() + fspecs,
        out_specs=[_flat_spec()] * SLC,
        out_shape=flat_out,
    )(t1f, t1f, t1f, t1f, t1f, t1f, *h1f, dis_f, *bg1f, *w2f)

    t2 = _conv_sc(*(h.reshape(NPo, SW) for h in h2), src_p, dst_p)
    t2f = t2.reshape(NC * SLC * NPo // 4, 4 * SW)

    outf = pl.pallas_call(
        _head_body,
        grid=(FGRID,),
        in_specs=_part_specs() + fspecs
        + [_full_spec((1, 4 * EMB)), _full_spec((4 * EMB, 8)),
           _full_spec((1, 8))],
        out_specs=pl.BlockSpec((FRB, 8), lambda i: (i, 0)),
        out_shape=jax.ShapeDtypeStruct((NPo // 4, 8), f32),
    )(t2f, t2f, t2f, t2f, t2f, t2f, *h2, dis_f, *bg2f, *wo1f,
      bo1f, wo2f, bo2f)
    return outf.reshape(NPo, 2)[:N]


# speedup vs baseline: 1.0227x; 1.0227x over previous
import functools

import jax
import jax.numpy as jnp
from jax import lax
from jax.experimental import pallas as pl
from jax.experimental.pallas import tpu as pltpu
from jax.experimental.pallas import tpu_sc as plsc

N = 50000
E = 800000
EMB = 96
H = EMB // 3

NC = 2
NS = 16
G = 128

SLC = 3
SW = EMB // SLC
SPT = N // NS
DST_PAD = N
NPo = 50176

E_PAD = 802816
EROWS = E_PAD // G
RPT_D = EROWS // (NC * NS)
SCH_R = 28
NSCH = RPT_D // SCH_R
GR = 2
NG2 = SCH_R // GR
GE = GR * G

NUP = 50176
ZSH = 51712
ZPT = ZSH // NS

_MESH = plsc.VectorSubcoreMesh(core_axis_name="c", subcore_axis_name="s")


def _leaky(x):
    return jnp.where(x > 0, x, 0.01 * x)


@functools.partial(
    pl.kernel,
    out_type=jax.ShapeDtypeStruct((NC * NUP,), jnp.float32),
    mesh=_MESH,
    scratch_types=[
        pltpu.VMEM_SHARED((ZSH,), jnp.float32),
        pltpu.VMEM((ZPT,), jnp.float32),
        pltpu.VMEM((G,), jnp.float32),
        pltpu.VMEM((4, G), jnp.int32),
    ],
)
def _deg_sc(dst_hbm, out_hbm, deg_sh, zbuf, oneb, idxb):
    c = lax.axis_index("c")
    s = lax.axis_index("s")
    for i in range(ZPT // 16):
        zbuf[pl.ds(i * 16, 16)] = jnp.zeros((16,), jnp.float32)
    pltpu.sync_copy(zbuf, deg_sh.at[pl.ds(s * ZPT, ZPT)])
    for i in range(G // 16):
        oneb[pl.ds(i * 16, 16)] = jnp.ones((16,), jnp.float32)
    plsc.subcore_barrier()

    base = (c * NS + s) * RPT_D
    def body(i, carry):
        pltpu.sync_copy(dst_hbm.at[pl.ds(base + i * 4, 4)], idxb)
        for j in range(4):
            pltpu.sync_copy(oneb, deg_sh.at[idxb.at[j]], add=True)
        return carry
    lax.fori_loop(0, RPT_D // 4, body, 0)
    plsc.subcore_barrier()

    span = NUP // NS
    pltpu.sync_copy(deg_sh.at[pl.ds(s * span, span)], zbuf.at[pl.ds(0, span)])
    pltpu.sync_copy(zbuf.at[pl.ds(0, span)],
                    out_hbm.at[pl.ds(c * NUP + s * span, span)])


@functools.partial(
    pl.kernel,
    out_type=jax.ShapeDtypeStruct((NC * SLC * NPo, SW), jnp.float32),
    mesh=_MESH,
    scratch_types=[
        pltpu.VMEM_SHARED((N + 8, SW), jnp.float32),
        pltpu.VMEM((SCH_R * G,), jnp.int32),
        pltpu.VMEM((SCH_R * G,), jnp.int32),
        pltpu.VMEM((GE, SW), jnp.float32),
        pltpu.VMEM((GE, SW), jnp.float32),
        pltpu.SemaphoreType.DMA,
        pltpu.SemaphoreType.DMA,
        pltpu.SemaphoreType.DMA,
    ],
    compiler_params=pltpu.CompilerParams(use_tc_tiling_on_sc=False),
)
def _conv_sc(hp0, hp1, hp2, src_hbm, dst_hbm, out_hbm,
             acc_sh, srcb, dstb, gb0, gb1, sem0, sem1, ssem):
    c = lax.axis_index("c")
    s = lax.axis_index("s")
    gbufs = (gb0, gb1)
    sems = (sem0, sem1)
    start_g = s * SPT
    rowbase = c * (EROWS // NC) + s * RPT_D

    for p, hp in enumerate((hp0, hp1, hp2)):
        pltpu.sync_copy(hp.at[pl.ds(start_g, SPT)],
                        acc_sh.at[pl.ds(start_g, SPT)])
        plsc.subcore_barrier()

        def schunk(si, carry):
            base = (rowbase + si * SCH_R) * G
            pltpu.sync_copy(src_hbm.at[pl.ds(base, SCH_R * G)], srcb)
            pltpu.sync_copy(dst_hbm.at[pl.ds(base, SCH_R * G)], dstb)
            gds = [None, None]
            gds[0] = pltpu.async_copy(
                hp.at[srcb.at[pl.ds(0, GE)]], gb0, sem0)
            for k in range(NG2):
                b = k % 2
                gds[b].wait()
                if k + 1 < NG2:
                    gds[1 - b] = pltpu.async_copy(
                        hp.at[srcb.at[pl.ds((k + 1) * GE, GE)]],
                        gbufs[1 - b], sems[1 - b])
                pltpu.async_copy(
                    gbufs[b], acc_sh.at[dstb.at[pl.ds(k * GE, GE)]],
                    ssem, add=True).wait()
            return carry
        lax.fori_loop(0, NSCH, schunk, 0)
        plsc.subcore_barrier()
        q = c * SLC + p
        pltpu.sync_copy(acc_sh.at[pl.ds(start_g, SPT)],
                        out_hbm.at[pl.ds(q * NPo + start_g, SPT)])
        plsc.subcore_barrier()


RB = 1000
GRID = N // RB


def _mm(a, b):
    return jnp.dot(a, b, preferred_element_type=jnp.float32)


def _enc_body(des_b, num_b, cat_b, Wd_b, bd_b, Wn_b, bn_b, Wc_b, bc_b,
              Wi_b, bi_b, out_b):
    d = _leaky(_mm(des_b[...], Wd_b[...]) + bd_b[...])
    n = _leaky(_mm(num_b[...], Wn_b[...]) + bn_b[...])
    cc = _leaky(_mm(cat_b[...], Wc_b[...]) + bc_b[...])
    x = jnp.concatenate([d, n, cc], axis=1)
    out_b[...] = _leaky(_mm(x, Wi_b[...]) + bi_b[...])


def _dis_of(degs_b):
    dd = degs_b[...]
    return lax.rsqrt(1.0 + dd[:, 0] + dd[:, 1])


def _split3(h, o0, o1, o2):
    o0[...] = h[:, 0:SW]
    o1[...] = h[:, SW:2 * SW]
    o2[...] = h[:, 2 * SW:EMB]


def _h1_body(x1f_b, d0f_b, d1f_b, W0, W1, W2, o0, o1, o2, od):
    df = lax.rsqrt(1.0 + d0f_b[...] + d1f_b[...])
    od[...] = df
    xf = x1f_b[...]
    o0[...] = _mm(xf, W0[...]) * df
    o1[...] = _mm(xf, W1[...]) * df
    o2[...] = _mm(xf, W2[...]) * df


def _pack_x(parts, hps, df, bf):
    return [
        (parts[p][...] + parts[SLC + p][...] - hps[p][...]) * df + bf[p][...]
        for p in range(SLC)]


def _yf_slice(yf, q, df):
    return jnp.concatenate(
        [yf[:, 96 * a + SW * q:96 * a + SW * q + SW] for a in range(4)],
        axis=1) * df


def _h2_body(a0, a1, a2, a3, a4, a5, h0, h1, h2, df_b, bf0, bf1, bf2,
             W0, W1, W2, o0, o1, o2):
    df = df_b[...]
    xfs = _pack_x((a0, a1, a2, a3, a4, a5), (h0, h1, h2), df,
                  (bf0, bf1, bf2))
    yf = _mm(xfs[0], W0[...]) + _mm(xfs[1], W1[...]) + _mm(xfs[2], W2[...])
    o0[...] = _yf_slice(yf, 0, df)
    o1[...] = _yf_slice(yf, 1, df)
    o2[...] = _yf_slice(yf, 2, df)


def _head_body(a0, a1, a2, a3, a4, a5, h0, h1, h2, df_b, bf0, bf1, bf2,
               W0, W1, W2, bo1f_b, Wo2f_b, bo2f_b, out_b):
    df = df_b[...]
    xfs = _pack_x((a0, a1, a2, a3, a4, a5), (h0, h1, h2), df,
                  (bf0, bf1, bf2))
    yf = _leaky(_mm(xfs[0], W0[...]) + _mm(xfs[1], W1[...])
                + _mm(xfs[2], W2[...]) + bo1f_b[...])
    out_b[...] = _mm(yf, Wo2f_b[...]) + bo2f_b[...]


def _row_spec(cols):
    return pl.BlockSpec((RB, cols), lambda i: (i, 0))


def _full_spec(shape):
    return pl.BlockSpec(shape, lambda i: tuple(0 for _ in shape))


def _deg_spec():
    return pl.BlockSpec((RB, 2), lambda i: (i, 0))


FRB = 256
FGRID = NPo // (4 * FRB)
FQ = NPo // (4 * FRB)


def _flat_spec():
    return pl.BlockSpec((FRB, 4 * SW), lambda i: (i, 0))


def _part_spec(q):
    return pl.BlockSpec((FRB, 4 * SW), lambda i, q=q: (q * FQ + i, 0))


def _part_specs():
    return [_part_spec(q) for q in range(NC * SLC)]


def _kron4(w):
    return jnp.kron(jnp.eye(4, dtype=w.dtype), w)


def kernel(des, tweet, num_prop, cat_prop, edge_index,
           Wd, bd, Wn, bn, Wc, bc, Wi, bi, Wg1, bg1, Wg2, bg2,
           Wo1, bo1, Wo2, bo2):
    del tweet
    f32 = jnp.float32
    src = edge_index[0]
    dst = edge_index[1]
    pad = E_PAD - E
    src_p = jnp.concatenate([src, jnp.zeros((pad,), jnp.int32)])
    dst_p = jnp.concatenate(
        [dst, jnp.full((pad,), DST_PAD, jnp.int32)])

    deg_raw = _deg_sc(dst_p.reshape(EROWS, G)).reshape(NC, NUP)

    def pack_f(v):
        return jnp.broadcast_to(
            v.reshape(NPo // 4, 4, 1),
            (NPo // 4, 4, SW)).reshape(NPo // 4, 4 * SW)

    d0f = pack_f(deg_raw[0])
    d1f = pack_f(deg_raw[1])

    x1 = pl.pallas_call(
        _enc_body,
        grid=(GRID,),
        in_specs=[
            _row_spec(768), _row_spec(4), _row_spec(3),
            _full_spec((768, H)), _full_spec((1, H)),
            _full_spec((4, H)), _full_spec((1, H)),
            _full_spec((3, H)), _full_spec((1, H)),
            _full_spec((EMB, EMB)), _full_spec((1, EMB)),
        ],
        out_specs=_row_spec(EMB),
        out_shape=jax.ShapeDtypeStruct((N, EMB), f32),
    )(des, num_prop, cat_prop, Wd, bd.reshape(1, H), Wn, bn.reshape(1, H),
      Wc, bc.reshape(1, H), Wi, bi.reshape(1, EMB))

    x1f = x1.reshape(N // 4, 4 * EMB)
    w1f = [_kron4(Wg1[:, SW * p:SW * (p + 1)]) for p in range(SLC)]
    h1out = pl.pallas_call(
        _h1_body,
        grid=(FGRID,),
        in_specs=[pl.BlockSpec((FRB, 4 * EMB), lambda i: (i, 0)),
                  _flat_spec(), _flat_spec()]
        + [_full_spec((4 * EMB, 4 * SW))] * SLC,
        out_specs=[_flat_spec()] * SLC + [_flat_spec()],
        out_shape=[jax.ShapeDtypeStruct((NPo // 4, 4 * SW), f32)] * (SLC + 1),
    )(x1f, d0f, d1f, *w1f)
    h1f, dis_f = list(h1out[:SLC]), h1out[SLC]

    w2f = [_kron4(Wg2[SW * p:SW * (p + 1), :]) for p in range(SLC)]
    wo1f = [_kron4(Wo1[SW * p:SW * (p + 1), :]) for p in range(SLC)]
    wo2f = _kron4(Wo2)
    bg1f = [jnp.tile(bg1[SW * p:SW * (p + 1)], 4).reshape(1, 4 * SW)
            for p in range(SLC)]
    bg2f = [jnp.tile(bg2[SW * p:SW * (p + 1)], 4).reshape(1, 4 * SW)
            for p in range(SLC)]
    bo1f = jnp.tile(bo1, 4).reshape(1, 4 * EMB)
    bo2f = jnp.tile(bo2, 4).reshape(1, 8)

    flat_out = [jax.ShapeDtypeStruct((NPo // 4, 4 * SW), f32)] * SLC
    fspecs = ([_flat_spec()] * SLC
              + [_flat_spec(), _full_spec((1, 4 * SW)),
                 _full_spec((1, 4 * SW)), _full_spec((1, 4 * SW)),
                 _full_spec((4 * SW, 4 * EMB)), _full_spec((4 * SW, 4 * EMB)),
                 _full_spec((4 * SW, 4 * EMB))])

    t1 = _conv_sc(*(h.reshape(NPo, SW) for h in h1f), src_p, dst_p)
    t1f = t1.reshape(NC * SLC * NPo // 4, 4 * SW)

    h2 = pl.pallas_call(
        _h2_body,
        grid=(FGRID,),
        in_specs=_part_specs() + fspecs,
        out_specs=[_flat_spec()] * SLC,
        out_shape=flat_out,
    )(t1f, t1f, t1f, t1f, t1f, t1f, *h1f, dis_f, *bg1f, *w2f)

    t2 = _conv_sc(*(h.reshape(NPo, SW) for h in h2), src_p, dst_p)
    t2f = t2.reshape(NC * SLC * NPo // 4, 4 * SW)

    outf = pl.pallas_call(
        _head_body,
        grid=(FGRID,),
        in_specs=_part_specs() + fspecs
        + [_full_spec((1, 4 * EMB)), _full_spec((4 * EMB, 8)),
           _full_spec((1, 8))],
        out_specs=pl.BlockSpec((FRB, 8), lambda i: (i, 0)),
        out_shape=jax.ShapeDtypeStruct((NPo // 4, 8), f32),
    )(t2f, t2f, t2f, t2f, t2f, t2f, *h2, dis_f, *bg2f, *wo1f,
      bo1f, wo2f, bo2f)
    return outf.reshape(NPo, 2)[:N]


# speedup vs baseline: 1.0482x; 1.0250x over previous
import functools

import jax
import jax.numpy as jnp
from jax import lax
from jax.experimental import pallas as pl
from jax.experimental.pallas import tpu as pltpu
from jax.experimental.pallas import tpu_sc as plsc

N = 50000
E = 800000
EMB = 96
H = EMB // 3

NC = 2
NS = 16
G = 128

SLC = 3
SW = EMB // SLC
SPT = N // NS
DST_PAD = N
NPo = 50176

E_PAD = 802816
EROWS = E_PAD // G
RPT_D = EROWS // (NC * NS)
SCH_R = 28
NSCH = RPT_D // SCH_R
GR = 2
NG2 = SCH_R // GR
GE = GR * G

NUP = 50176
ZSH = 51712
ZPT = ZSH // NS

_MESH = plsc.VectorSubcoreMesh(core_axis_name="c", subcore_axis_name="s")


def _leaky(x):
    return jnp.where(x > 0, x, 0.01 * x)


@functools.partial(
    pl.kernel,
    out_type=jax.ShapeDtypeStruct((NC * NUP,), jnp.float32),
    mesh=_MESH,
    scratch_types=[
        pltpu.VMEM_SHARED((ZSH,), jnp.float32),
        pltpu.VMEM((ZPT,), jnp.float32),
        pltpu.VMEM((4 * G,), jnp.float32),
        pltpu.VMEM((SCH_R * G,), jnp.int32),
    ],
)
def _deg_sc(dst_hbm, out_hbm, deg_sh, zbuf, oneb, idxb):
    c = lax.axis_index("c")
    s = lax.axis_index("s")
    for i in range(ZPT // 16):
        zbuf[pl.ds(i * 16, 16)] = jnp.zeros((16,), jnp.float32)
    pltpu.sync_copy(zbuf, deg_sh.at[pl.ds(s * ZPT, ZPT)])
    for i in range(4 * G // 16):
        oneb[pl.ds(i * 16, 16)] = jnp.ones((16,), jnp.float32)
    plsc.subcore_barrier()

    base = (c * NS + s) * RPT_D
    def body(i, carry):
        pltpu.sync_copy(dst_hbm.at[pl.ds((base + i * SCH_R) * G, SCH_R * G)],
                        idxb)
        for j in range(SCH_R // 4):
            pltpu.sync_copy(oneb, deg_sh.at[idxb.at[pl.ds(j * 4 * G, 4 * G)]],
                            add=True)
        return carry
    lax.fori_loop(0, RPT_D // SCH_R, body, 0)
    plsc.subcore_barrier()

    span = NUP // NS
    pltpu.sync_copy(deg_sh.at[pl.ds(s * span, span)], zbuf.at[pl.ds(0, span)])
    pltpu.sync_copy(zbuf.at[pl.ds(0, span)],
                    out_hbm.at[pl.ds(c * NUP + s * span, span)])


@functools.partial(
    pl.kernel,
    out_type=jax.ShapeDtypeStruct((NC * SLC * NPo, SW), jnp.float32),
    mesh=_MESH,
    scratch_types=[
        pltpu.VMEM_SHARED((N + 8, SW), jnp.float32),
        pltpu.VMEM((SCH_R * G,), jnp.int32),
        pltpu.VMEM((SCH_R * G,), jnp.int32),
        pltpu.VMEM((GE, SW), jnp.float32),
        pltpu.VMEM((GE, SW), jnp.float32),
        pltpu.SemaphoreType.DMA,
        pltpu.SemaphoreType.DMA,
        pltpu.SemaphoreType.DMA,
    ],
    compiler_params=pltpu.CompilerParams(use_tc_tiling_on_sc=False),
)
def _conv_sc(hp0, hp1, hp2, src_hbm, dst_hbm, out_hbm,
             acc_sh, srcb, dstb, gb0, gb1, sem0, sem1, ssem):
    c = lax.axis_index("c")
    s = lax.axis_index("s")
    gbufs = (gb0, gb1)
    sems = (sem0, sem1)
    start_g = s * SPT
    rowbase = c * (EROWS // NC) + s * RPT_D

    for p, hp in enumerate((hp0, hp1, hp2)):
        pltpu.sync_copy(hp.at[pl.ds(start_g, SPT)],
                        acc_sh.at[pl.ds(start_g, SPT)])
        plsc.subcore_barrier()

        def schunk(si, carry):
            base = (rowbase + si * SCH_R) * G
            pltpu.sync_copy(src_hbm.at[pl.ds(base, SCH_R * G)], srcb)
            pltpu.sync_copy(dst_hbm.at[pl.ds(base, SCH_R * G)], dstb)
            gds = [None, None]
            gds[0] = pltpu.async_copy(
                hp.at[srcb.at[pl.ds(0, GE)]], gb0, sem0)
            for k in range(NG2):
                b = k % 2
                gds[b].wait()
                if k + 1 < NG2:
                    gds[1 - b] = pltpu.async_copy(
                        hp.at[srcb.at[pl.ds((k + 1) * GE, GE)]],
                        gbufs[1 - b], sems[1 - b])
                pltpu.async_copy(
                    gbufs[b], acc_sh.at[dstb.at[pl.ds(k * GE, GE)]],
                    ssem, add=True).wait()
            return carry
        lax.fori_loop(0, NSCH, schunk, 0)
        plsc.subcore_barrier()
        q = c * SLC + p
        pltpu.sync_copy(acc_sh.at[pl.ds(start_g, SPT)],
                        out_hbm.at[pl.ds(q * NPo + start_g, SPT)])
        plsc.subcore_barrier()


RB = 1000
GRID = N // RB


def _mm(a, b):
    return jnp.dot(a, b, preferred_element_type=jnp.float32)


def _enc_body(des_b, num_b, cat_b, Wd_b, bd_b, Wn_b, bn_b, Wc_b, bc_b,
              Wi_b, bi_b, out_b):
    d = _leaky(_mm(des_b[...], Wd_b[...]) + bd_b[...])
    n = _leaky(_mm(num_b[...], Wn_b[...]) + bn_b[...])
    cc = _leaky(_mm(cat_b[...], Wc_b[...]) + bc_b[...])
    x = jnp.concatenate([d, n, cc], axis=1)
    out_b[...] = _leaky(_mm(x, Wi_b[...]) + bi_b[...])


def _dis_of(degs_b):
    dd = degs_b[...]
    return lax.rsqrt(1.0 + dd[:, 0] + dd[:, 1])


def _split3(h, o0, o1, o2):
    o0[...] = h[:, 0:SW]
    o1[...] = h[:, SW:2 * SW]
    o2[...] = h[:, 2 * SW:EMB]


def _h1_body(x1f_b, d0f_b, d1f_b, W0, W1, W2, o0, o1, o2, od):
    df = lax.rsqrt(1.0 + d0f_b[0] + d1f_b[0])
    od[...] = df
    xf = x1f_b[...]
    o0[...] = _mm(xf, W0[...]) * df
    o1[...] = _mm(xf, W1[...]) * df
    o2[...] = _mm(xf, W2[...]) * df


def _pack_x(parts, hps, df, bf):
    return [
        (parts[p][...] + parts[SLC + p][...] - hps[p][...]) * df + bf[p][...]
        for p in range(SLC)]


def _yf_slice(yf, q, df):
    return jnp.concatenate(
        [yf[:, 96 * a + SW * q:96 * a + SW * q + SW] for a in range(4)],
        axis=1) * df


def _h2_body(a0, a1, a2, a3, a4, a5, h0, h1, h2, df_b, bf0, bf1, bf2,
             W0, W1, W2, o0, o1, o2):
    df = df_b[...]
    xfs = _pack_x((a0, a1, a2, a3, a4, a5), (h0, h1, h2), df,
                  (bf0, bf1, bf2))
    yf = _mm(xfs[0], W0[...]) + _mm(xfs[1], W1[...]) + _mm(xfs[2], W2[...])
    o0[...] = _yf_slice(yf, 0, df)
    o1[...] = _yf_slice(yf, 1, df)
    o2[...] = _yf_slice(yf, 2, df)


def _head_body(a0, a1, a2, a3, a4, a5, h0, h1, h2, df_b, bf0, bf1, bf2,
               W0, W1, W2, bo1f_b, Wo2f_b, bo2f_b, out_b):
    df = df_b[...]
    xfs = _pack_x((a0, a1, a2, a3, a4, a5), (h0, h1, h2), df,
                  (bf0, bf1, bf2))
    yf = _leaky(_mm(xfs[0], W0[...]) + _mm(xfs[1], W1[...])
                + _mm(xfs[2], W2[...]) + bo1f_b[...])
    out_b[...] = _mm(yf, Wo2f_b[...]) + bo2f_b[...]


def _row_spec(cols):
    return pl.BlockSpec((RB, cols), lambda i: (i, 0))


def _full_spec(shape):
    return pl.BlockSpec(shape, lambda i: tuple(0 for _ in shape))


def _deg_spec():
    return pl.BlockSpec((RB, 2), lambda i: (i, 0))


FRB = 256
FGRID = NPo // (4 * FRB)
FQ = NPo // (4 * FRB)


def _flat_spec():
    return pl.BlockSpec((FRB, 4 * SW), lambda i: (i, 0))


def _part_spec(q):
    return pl.BlockSpec((FRB, 4 * SW), lambda i, q=q: (q * FQ + i, 0))


def _part_specs():
    return [_part_spec(q) for q in range(NC * SLC)]


def _kron4(w):
    return jnp.kron(jnp.eye(4, dtype=w.dtype), w)


def kernel(des, tweet, num_prop, cat_prop, edge_index,
           Wd, bd, Wn, bn, Wc, bc, Wi, bi, Wg1, bg1, Wg2, bg2,
           Wo1, bo1, Wo2, bo2):
    del tweet
    f32 = jnp.float32
    src = edge_index[0]
    dst = edge_index[1]
    pad = E_PAD - E
    src_p = jnp.concatenate([src, jnp.zeros((pad,), jnp.int32)])
    dst_p = jnp.concatenate(
        [dst, jnp.full((pad,), DST_PAD, jnp.int32)])

    deg_raw = _deg_sc(dst_p)
    d01f = jnp.broadcast_to(
        deg_raw.reshape(NC, NPo // 4, 4, 1),
        (NC, NPo // 4, 4, SW)).reshape(NC, NPo // 4, 4 * SW)

    x1 = pl.pallas_call(
        _enc_body,
        grid=(GRID,),
        in_specs=[
            _row_spec(768), _row_spec(4), _row_spec(3),
            _full_spec((768, H)), _full_spec((1, H)),
            _full_spec((4, H)), _full_spec((1, H)),
            _full_spec((3, H)), _full_spec((1, H)),
            _full_spec((EMB, EMB)), _full_spec((1, EMB)),
        ],
        out_specs=_row_spec(EMB),
        out_shape=jax.ShapeDtypeStruct((N, EMB), f32),
    )(des, num_prop, cat_prop, Wd, bd.reshape(1, H), Wn, bn.reshape(1, H),
      Wc, bc.reshape(1, H), Wi, bi.reshape(1, EMB))

    x1f = x1.reshape(N // 4, 4 * EMB)
    w1f = [_kron4(Wg1[:, SW * p:SW * (p + 1)]) for p in range(SLC)]
    h1out = pl.pallas_call(
        _h1_body,
        grid=(FGRID,),
        in_specs=[pl.BlockSpec((FRB, 4 * EMB), lambda i: (i, 0)),
                  pl.BlockSpec((1, FRB, 4 * SW), lambda i: (0, i, 0)),
                  pl.BlockSpec((1, FRB, 4 * SW), lambda i: (1, i, 0))]
        + [_full_spec((4 * EMB, 4 * SW))] * SLC,
        out_specs=[_flat_spec()] * SLC + [_flat_spec()],
        out_shape=[jax.ShapeDtypeStruct((NPo // 4, 4 * SW), f32)] * (SLC + 1),
    )(x1f, d01f, d01f, *w1f)
    h1f, dis_f = list(h1out[:SLC]), h1out[SLC]

    w2f = [_kron4(Wg2[SW * p:SW * (p + 1), :]) for p in range(SLC)]
    wo1f = [_kron4(Wo1[SW * p:SW * (p + 1), :]) for p in range(SLC)]
    wo2f = _kron4(Wo2)
    bg1f = [jnp.tile(bg1[SW * p:SW * (p + 1)], 4).reshape(1, 4 * SW)
            for p in range(SLC)]
    bg2f = [jnp.tile(bg2[SW * p:SW * (p + 1)], 4).reshape(1, 4 * SW)
            for p in range(SLC)]
    bo1f = jnp.tile(bo1, 4).reshape(1, 4 * EMB)
    bo2f = jnp.tile(bo2, 4).reshape(1, 8)

    flat_out = [jax.ShapeDtypeStruct((NPo // 4, 4 * SW), f32)] * SLC
    fspecs = ([_flat_spec()] * SLC
              + [_flat_spec(), _full_spec((1, 4 * SW)),
                 _full_spec((1, 4 * SW)), _full_spec((1, 4 * SW)),
                 _full_spec((4 * SW, 4 * EMB)), _full_spec((4 * SW, 4 * EMB)),
                 _full_spec((4 * SW, 4 * EMB))])

    t1 = _conv_sc(*(h.reshape(NPo, SW) for h in h1f), src_p, dst_p)
    t1f = t1.reshape(NC * SLC * NPo // 4, 4 * SW)

    h2 = pl.pallas_call(
        _h2_body,
        grid=(FGRID,),
        in_specs=_part_specs() + fspecs,
        out_specs=[_flat_spec()] * SLC,
        out_shape=flat_out,
    )(t1f, t1f, t1f, t1f, t1f, t1f, *h1f, dis_f, *bg1f, *w2f)

    t2 = _conv_sc(*(h.reshape(NPo, SW) for h in h2), src_p, dst_p)
    t2f = t2.reshape(NC * SLC * NPo // 4, 4 * SW)

    outf = pl.pallas_call(
        _head_body,
        grid=(FGRID,),
        in_specs=_part_specs() + fspecs
        + [_full_spec((1, 4 * EMB)), _full_spec((4 * EMB, 8)),
           _full_spec((1, 8))],
        out_specs=pl.BlockSpec((FRB, 8), lambda i: (i, 0)),
        out_shape=jax.ShapeDtypeStruct((NPo // 4, 8), f32),
    )(t2f, t2f, t2f, t2f, t2f, t2f, *h2, dis_f, *bg2f, *wo1f,
      bo1f, wo2f, bo2f)
    return outf.reshape(NPo, 2)[:N]


# speedup vs baseline: 1.0899x; 1.0398x over previous
import functools

import jax
import jax.numpy as jnp
from jax import lax
from jax.experimental import pallas as pl
from jax.experimental.pallas import tpu as pltpu
from jax.experimental.pallas import tpu_sc as plsc

N = 50000
E = 800000
EMB = 96
H = EMB // 3

NC = 2
NS = 16
G = 128

SLC = 3
SW = EMB // SLC
SPT = N // NS
DST_PAD = N
NPo = 50176

E_PAD = 802816
EROWS = E_PAD // G
RPT_D = EROWS // (NC * NS)
SCH_R = 28
NSCH = RPT_D // SCH_R
GR = 2
NG2 = SCH_R // GR
GE = GR * G

NUP = 50176
ZSH = 51712
ZPT = ZSH // NS

_MESH = plsc.VectorSubcoreMesh(core_axis_name="c", subcore_axis_name="s")


def _leaky(x):
    return jnp.where(x > 0, x, 0.01 * x)


@functools.partial(
    pl.kernel,
    out_type=jax.ShapeDtypeStruct((NC * NUP,), jnp.float32),
    mesh=_MESH,
    scratch_types=[
        pltpu.VMEM_SHARED((ZSH,), jnp.float32),
        pltpu.VMEM((ZPT,), jnp.float32),
        pltpu.VMEM((4 * G,), jnp.float32),
        pltpu.VMEM((SCH_R * G,), jnp.int32),
    ],
)
def _deg_sc(dst_hbm, out_hbm, deg_sh, zbuf, oneb, idxb):
    c = lax.axis_index("c")
    s = lax.axis_index("s")
    for i in range(ZPT // 16):
        zbuf[pl.ds(i * 16, 16)] = jnp.zeros((16,), jnp.float32)
    pltpu.sync_copy(zbuf, deg_sh.at[pl.ds(s * ZPT, ZPT)])
    for i in range(4 * G // 16):
        oneb[pl.ds(i * 16, 16)] = jnp.ones((16,), jnp.float32)
    plsc.subcore_barrier()

    base = (c * NS + s) * RPT_D
    def body(i, carry):
        pltpu.sync_copy(dst_hbm.at[pl.ds((base + i * SCH_R) * G, SCH_R * G)],
                        idxb)
        for j in range(SCH_R // 4):
            pltpu.sync_copy(oneb, deg_sh.at[idxb.at[pl.ds(j * 4 * G, 4 * G)]],
                            add=True)
        return carry
    lax.fori_loop(0, RPT_D // SCH_R, body, 0)
    plsc.subcore_barrier()

    span = NUP // NS
    pltpu.sync_copy(deg_sh.at[pl.ds(s * span, span)], zbuf.at[pl.ds(0, span)])
    pltpu.sync_copy(zbuf.at[pl.ds(0, span)],
                    out_hbm.at[pl.ds(c * NUP + s * span, span)])


@functools.partial(
    pl.kernel,
    out_type=jax.ShapeDtypeStruct((NC * SLC * NPo, SW), jnp.float32),
    mesh=_MESH,
    scratch_types=[
        pltpu.VMEM_SHARED((N + 8, SW), jnp.float32),
        pltpu.VMEM((SCH_R * G,), jnp.int32),
        pltpu.VMEM((SCH_R * G,), jnp.int32),
        pltpu.VMEM((SCH_R * G,), jnp.int32),
        pltpu.VMEM((SCH_R * G,), jnp.int32),
        pltpu.VMEM((GE, SW), jnp.float32),
        pltpu.VMEM((GE, SW), jnp.float32),
        pltpu.SemaphoreType.DMA,
        pltpu.SemaphoreType.DMA,
        pltpu.SemaphoreType.DMA,
        pltpu.SemaphoreType.DMA,
        pltpu.SemaphoreType.DMA,
    ],
    compiler_params=pltpu.CompilerParams(use_tc_tiling_on_sc=False),
)
def _conv_sc(hp0, hp1, hp2, src_hbm, dst_hbm, out_hbm,
             acc_sh, srcb0, dstb0, srcb1, dstb1, gb0, gb1,
             sem0, sem1, ssem, stg0, stg1):
    c = lax.axis_index("c")
    s = lax.axis_index("s")
    gbufs = (gb0, gb1)
    sems = (sem0, sem1)
    srcbs = (srcb0, srcb1)
    dstbs = (dstb0, dstb1)
    stgs = (stg0, stg1)
    start_g = s * SPT
    rowbase = c * (EROWS // NC) + s * RPT_D

    def stage(si, b):
        base = (rowbase + si * SCH_R) * G
        return (pltpu.async_copy(src_hbm.at[pl.ds(base, SCH_R * G)],
                                 srcbs[b], stgs[b]),
                pltpu.async_copy(dst_hbm.at[pl.ds(base, SCH_R * G)],
                                 dstbs[b], stgs[b]))

    for p, hp in enumerate((hp0, hp1, hp2)):
        pltpu.sync_copy(hp.at[pl.ds(start_g, SPT)],
                        acc_sh.at[pl.ds(start_g, SPT)])
        plsc.subcore_barrier()

        sds = [None, None]
        sds[0] = stage(0, 0)
        for si in range(NSCH):
            sb = si % 2
            srcb, dstb = srcbs[sb], dstbs[sb]
            for d in sds[sb]:
                d.wait()
            if si + 1 < NSCH:
                sds[1 - sb] = stage(si + 1, 1 - sb)
            gds = [None, None]
            gds[0] = pltpu.async_copy(
                hp.at[srcb.at[pl.ds(0, GE)]], gb0, sem0)
            for k in range(NG2):
                b = k % 2
                gds[b].wait()
                if k + 1 < NG2:
                    gds[1 - b] = pltpu.async_copy(
                        hp.at[srcb.at[pl.ds((k + 1) * GE, GE)]],
                        gbufs[1 - b], sems[1 - b])
                pltpu.async_copy(
                    gbufs[b], acc_sh.at[dstb.at[pl.ds(k * GE, GE)]],
                    ssem, add=True).wait()
        plsc.subcore_barrier()
        q = c * SLC + p
        pltpu.sync_copy(acc_sh.at[pl.ds(start_g, SPT)],
                        out_hbm.at[pl.ds(q * NPo + start_g, SPT)])
        plsc.subcore_barrier()


RB = 1000
GRID = N // RB


def _mm(a, b):
    return jnp.dot(a, b, preferred_element_type=jnp.float32)


def _enc_body(des_b, num_b, cat_b, Wd_b, bd_b, Wn_b, bn_b, Wc_b, bc_b,
              Wi_b, bi_b, out_b):
    d = _leaky(_mm(des_b[...], Wd_b[...]) + bd_b[...])
    n = _leaky(_mm(num_b[...], Wn_b[...]) + bn_b[...])
    cc = _leaky(_mm(cat_b[...], Wc_b[...]) + bc_b[...])
    x = jnp.concatenate([d, n, cc], axis=1)
    out_b[...] = _leaky(_mm(x, Wi_b[...]) + bi_b[...])


def _dis_of(degs_b):
    dd = degs_b[...]
    return lax.rsqrt(1.0 + dd[:, 0] + dd[:, 1])


def _split3(h, o0, o1, o2):
    o0[...] = h[:, 0:SW]
    o1[...] = h[:, SW:2 * SW]
    o2[...] = h[:, 2 * SW:EMB]


def _h1_body(x1f_b, d0f_b, d1f_b, W0, W1, W2, o0, o1, o2, od):
    df = lax.rsqrt(1.0 + d0f_b[0] + d1f_b[0])
    od[...] = df
    xf = x1f_b[...]
    o0[...] = _mm(xf, W0[...]) * df
    o1[...] = _mm(xf, W1[...]) * df
    o2[...] = _mm(xf, W2[...]) * df


def _pack_x(parts, hps, df, bf):
    return [
        (parts[p][...] + parts[SLC + p][...] - hps[p][...]) * df + bf[p][...]
        for p in range(SLC)]


def _yf_slice(yf, q, df):
    return jnp.concatenate(
        [yf[:, 96 * a + SW * q:96 * a + SW * q + SW] for a in range(4)],
        axis=1) * df


def _h2_body(a0, a1, a2, a3, a4, a5, h0, h1, h2, df_b, bf0, bf1, bf2,
             W0, W1, W2, o0, o1, o2):
    df = df_b[...]
    xfs = _pack_x((a0, a1, a2, a3, a4, a5), (h0, h1, h2), df,
                  (bf0, bf1, bf2))
    yf = _mm(xfs[0], W0[...]) + _mm(xfs[1], W1[...]) + _mm(xfs[2], W2[...])
    o0[...] = _yf_slice(yf, 0, df)
    o1[...] = _yf_slice(yf, 1, df)
    o2[...] = _yf_slice(yf, 2, df)


def _head_body(a0, a1, a2, a3, a4, a5, h0, h1, h2, df_b, bf0, bf1, bf2,
               W0, W1, W2, bo1f_b, Wo2f_b, bo2f_b, out_b):
    df = df_b[...]
    xfs = _pack_x((a0, a1, a2, a3, a4, a5), (h0, h1, h2), df,
                  (bf0, bf1, bf2))
    yf = _leaky(_mm(xfs[0], W0[...]) + _mm(xfs[1], W1[...])
                + _mm(xfs[2], W2[...]) + bo1f_b[...])
    out_b[...] = _mm(yf, Wo2f_b[...]) + bo2f_b[...]


def _row_spec(cols):
    return pl.BlockSpec((RB, cols), lambda i: (i, 0))


def _full_spec(shape):
    return pl.BlockSpec(shape, lambda i: tuple(0 for _ in shape))


def _deg_spec():
    return pl.BlockSpec((RB, 2), lambda i: (i, 0))


FRB = 256
FGRID = NPo // (4 * FRB)
FQ = NPo // (4 * FRB)


def _flat_spec():
    return pl.BlockSpec((FRB, 4 * SW), lambda i: (i, 0))


def _part_spec(q):
    return pl.BlockSpec((FRB, 4 * SW), lambda i, q=q: (q * FQ + i, 0))


def _part_specs():
    return [_part_spec(q) for q in range(NC * SLC)]


def _kron4(w):
    return jnp.kron(jnp.eye(4, dtype=w.dtype), w)


def kernel(des, tweet, num_prop, cat_prop, edge_index,
           Wd, bd, Wn, bn, Wc, bc, Wi, bi, Wg1, bg1, Wg2, bg2,
           Wo1, bo1, Wo2, bo2):
    del tweet
    f32 = jnp.float32
    src = edge_index[0]
    dst = edge_index[1]
    pad = E_PAD - E
    src_p = jnp.concatenate([src, jnp.zeros((pad,), jnp.int32)])
    dst_p = jnp.concatenate(
        [dst, jnp.full((pad,), DST_PAD, jnp.int32)])

    deg_raw = _deg_sc(dst_p)
    d01f = jnp.broadcast_to(
        deg_raw.reshape(NC, NPo // 4, 4, 1),
        (NC, NPo // 4, 4, SW)).reshape(NC, NPo // 4, 4 * SW)

    x1 = pl.pallas_call(
        _enc_body,
        grid=(GRID,),
        in_specs=[
            _row_spec(768), _row_spec(4), _row_spec(3),
            _full_spec((768, H)), _full_spec((1, H)),
            _full_spec((4, H)), _full_spec((1, H)),
            _full_spec((3, H)), _full_spec((1, H)),
            _full_spec((EMB, EMB)), _full_spec((1, EMB)),
        ],
        out_specs=_row_spec(EMB),
        out_shape=jax.ShapeDtypeStruct((N, EMB), f32),
    )(des, num_prop, cat_prop, Wd, bd.reshape(1, H), Wn, bn.reshape(1, H),
      Wc, bc.reshape(1, H), Wi, bi.reshape(1, EMB))

    x1f = x1.reshape(N // 4, 4 * EMB)
    w1f = [_kron4(Wg1[:, SW * p:SW * (p + 1)]) for p in range(SLC)]
    h1out = pl.pallas_call(
        _h1_body,
        grid=(FGRID,),
        in_specs=[pl.BlockSpec((FRB, 4 * EMB), lambda i: (i, 0)),
                  pl.BlockSpec((1, FRB, 4 * SW), lambda i: (0, i, 0)),
                  pl.BlockSpec((1, FRB, 4 * SW), lambda i: (1, i, 0))]
        + [_full_spec((4 * EMB, 4 * SW))] * SLC,
        out_specs=[_flat_spec()] * SLC + [_flat_spec()],
        out_shape=[jax.ShapeDtypeStruct((NPo // 4, 4 * SW), f32)] * (SLC + 1),
    )(x1f, d01f, d01f, *w1f)
    h1f, dis_f = list(h1out[:SLC]), h1out[SLC]

    w2f = [_kron4(Wg2[SW * p:SW * (p + 1), :]) for p in range(SLC)]
    wo1f = [_kron4(Wo1[SW * p:SW * (p + 1), :]) for p in range(SLC)]
    wo2f = _kron4(Wo2)
    bg1f = [jnp.tile(bg1[SW * p:SW * (p + 1)], 4).reshape(1, 4 * SW)
            for p in range(SLC)]
    bg2f = [jnp.tile(bg2[SW * p:SW * (p + 1)], 4).reshape(1, 4 * SW)
            for p in range(SLC)]
    bo1f = jnp.tile(bo1, 4).reshape(1, 4 * EMB)
    bo2f = jnp.tile(bo2, 4).reshape(1, 8)

    flat_out = [jax.ShapeDtypeStruct((NPo // 4, 4 * SW), f32)] * SLC
    fspecs = ([_flat_spec()] * SLC
              + [_flat_spec(), _full_spec((1, 4 * SW)),
                 _full_spec((1, 4 * SW)), _full_spec((1, 4 * SW)),
                 _full_spec((4 * SW, 4 * EMB)), _full_spec((4 * SW, 4 * EMB)),
                 _full_spec((4 * SW, 4 * EMB))])

    t1 = _conv_sc(*(h.reshape(NPo, SW) for h in h1f), src_p, dst_p)
    t1f = t1.reshape(NC * SLC * NPo // 4, 4 * SW)

    h2 = pl.pallas_call(
        _h2_body,
        grid=(FGRID,),
        in_specs=_part_specs() + fspecs,
        out_specs=[_flat_spec()] * SLC,
        out_shape=flat_out,
    )(t1f, t1f, t1f, t1f, t1f, t1f, *h1f, dis_f, *bg1f, *w2f)

    t2 = _conv_sc(*(h.reshape(NPo, SW) for h in h2), src_p, dst_p)
    t2f = t2.reshape(NC * SLC * NPo // 4, 4 * SW)

    outf = pl.pallas_call(
        _head_body,
        grid=(FGRID,),
        in_specs=_part_specs() + fspecs
        + [_full_spec((1, 4 * EMB)), _full_spec((4 * EMB, 8)),
           _full_spec((1, 8))],
        out_specs=pl.BlockSpec((FRB, 8), lambda i: (i, 0)),
        out_shape=jax.ShapeDtypeStruct((NPo // 4, 8), f32),
    )(t2f, t2f, t2f, t2f, t2f, t2f, *h2, dis_f, *bg2f, *wo1f,
      bo1f, wo2f, bo2f)
    return outf.reshape(NPo, 2)[:N]
